# Initial kernel scaffold; baseline (speedup 1.0000x reference)
import jax
import jax.numpy as jnp
from jax.experimental import pallas as pl


def kernel(x, input_pts, params):
    raise NotImplementedError("write your pallas kernel here")


# speedup vs baseline: 1.0001x; 1.0001x over previous
import functools

import jax
import jax.numpy as jnp
import numpy as np
from jax.experimental import pallas as pl
from jax.experimental.pallas import tpu as pltpu

PL = 64
NC = 16
DIM = 3
IN_CH = 3
OUT_CH = 13
B = 2
N = 8192
NPTS = [2048, 1024, 256, 64, 16, 8]
KSPEC = [("cv0", 0, 0, 16), ("cv1", 0, 1, 16), ("cv2", 1, 2, 16),
         ("cv3", 2, 3, 16), ("cv4", 3, 4, 8), ("cv5", 4, 5, 8),
         ("cv6", 5, 6, 4), ("cv5d", 6, 5, 4), ("cv4d", 5, 4, 4),
         ("cv3d", 4, 3, 4), ("cv2d", 3, 2, 8), ("cv1d", 2, 1, 8),
         ("cv0d", 1, 0, 8)]


def _pts_pyramid(input_pts):
    pts = [input_pts]
    for npts in NPTS:
        n = pts[-1].shape[1]
        sel = jnp.arange(npts) * (n // npts)
        pts.append(pts[-1][:, sel])
    return pts


def _knn(q, r, K):
    qn = (q ** 2).sum(-1)[:, :, None]
    rn = (r ** 2).sum(-1)[:, None, :]
    d = qn + rn - 2.0 * jnp.einsum("bmd,bnd->bmn", q, r)
    _, idx = jax.lax.top_k(-d, K)
    return idx


def _gather(a, idx):
    return jax.vmap(lambda ab, ib: ab[ib])(a, idx)


def _ptconv(p, x, pin, pout, idx, K):
    feats = _gather(x, idx)
    pts = _gather(pin, idx) - pout[:, :, None, :]
    maxi = jnp.sqrt(jax.lax.stop_gradient((pts ** 2).sum(-1)).max(-1))
    maxi = jnp.where(maxi == 0.0, 1.0, maxi)
    pts = pts / maxi[:, :, None, None]
    d = pts[..., None] - p["c"][None, None, None, :, :]
    d = d.reshape(d.shape[0], d.shape[1], d.shape[2], DIM * NC)
    d = jax.nn.relu(d @ p["l1w"] + p["l1b"])
    d = jax.nn.relu(d @ p["l2w"] + p["l2b"])
    d = jax.nn.relu(d @ p["l3w"] + p["l3b"])
    r = jnp.einsum("bmkc,bmkn->bmcn", feats, d)
    r = r.reshape(r.shape[0], r.shape[1], -1)
    return (r @ p["w"].reshape(-1, p["w"].shape[2])) / K


def _bn(x, bp, eps=1e-5):
    m = x.mean(axis=(0, 1))
    v = x.var(axis=(0, 1))
    return bp["g"] * (x - m) / jnp.sqrt(v + eps) + bp["b"]


def _fc_kernel(x_ref, w_ref, b_ref, o_ref):
    o_ref[...] = jnp.dot(x_ref[...], w_ref[...],
                         preferred_element_type=jnp.float32) + b_ref[...]


def _fc(x2d, w, b):
    R = x2d.shape[0]
    return pl.pallas_call(
        _fc_kernel,
        out_shape=jax.ShapeDtypeStruct((R, w.shape[1]), jnp.float32),
        in_specs=[pl.BlockSpec((R, x2d.shape[1]), lambda: (0, 0)),
                  pl.BlockSpec(w.shape, lambda: (0, 0)),
                  pl.BlockSpec((1, w.shape[1]), lambda: (0, 0))],
        out_specs=pl.BlockSpec((R, w.shape[1]), lambda: (0, 0)),
    )(x2d, w, b.reshape(1, -1))


def kernel(x, input_pts, params):
    pts = _pts_pyramid(input_pts)
    idx = {name: _knn(pts[o], pts[r], K) for (name, r, o, K) in KSPEC}

    def cv(name, xin, r, o, K):
        return jax.nn.relu(_bn(
            _ptconv(params[name], xin, pts[r], pts[o], idx[name], K),
            params["bn_" + name]))

    x0 = cv("cv0", x, 0, 0, 16)
    x1 = cv("cv1", x0, 0, 1, 16)
    x2 = cv("cv2", x1, 1, 2, 16)
    x3 = cv("cv3", x2, 2, 3, 16)
    x4 = cv("cv4", x3, 3, 4, 8)
    x5 = cv("cv5", x4, 4, 5, 8)
    x6 = cv("cv6", x5, 5, 6, 4)
    x5d = jnp.concatenate([cv("cv5d", x6, 6, 5, 4), x5], axis=2)
    x4d = jnp.concatenate([cv("cv4d", x5d, 5, 4, 4), x4], axis=2)
    x3d = jnp.concatenate([cv("cv3d", x4d, 4, 3, 4), x3], axis=2)
    x2d = jnp.concatenate([cv("cv2d", x3d, 3, 2, 8), x2], axis=2)
    x1d = jnp.concatenate([cv("cv1d", x2d, 2, 1, 8), x1], axis=2)
    x0d = jnp.concatenate([cv("cv0d", x1d, 1, 0, 8), x0], axis=2)

    xout = _fc(x0d.reshape(-1, x0d.shape[2]), params["fcout_w"],
               params["fcout_b"])
    xout = xout.reshape(x.shape[0], -1, xout.shape[1])
    return (xout, x0d)


# speedup vs baseline: 3.2242x; 3.2239x over previous
import functools

import jax
import jax.numpy as jnp
import numpy as np
from jax.experimental import pallas as pl
from jax.experimental.pallas import tpu as pltpu

PL = 64
NC = 16
DIM = 3
IN_CH = 3
OUT_CH = 13
B = 2
N = 8192
NPTS = [2048, 1024, 256, 64, 16, 8]
KSPEC = [("cv0", 0, 0, 16), ("cv1", 0, 1, 16), ("cv2", 1, 2, 16),
         ("cv3", 2, 3, 16), ("cv4", 3, 4, 8), ("cv5", 4, 5, 8),
         ("cv6", 5, 6, 4), ("cv5d", 6, 5, 4), ("cv4d", 5, 4, 4),
         ("cv3d", 4, 3, 4), ("cv2d", 3, 2, 8), ("cv1d", 2, 1, 8),
         ("cv0d", 1, 0, 8)]


def _pts_pyramid(input_pts):
    pts = [input_pts]
    for npts in NPTS:
        n = pts[-1].shape[1]
        sel = jnp.arange(npts) * (n // npts)
        pts.append(pts[-1][:, sel])
    return pts


def _knn_body(K, NR, q_ref, r_ref, o_ref):
    q = q_ref[0]
    r = r_ref[0]
    qn = jnp.sum(q * q, axis=1, keepdims=True)
    rn = jax.lax.dot_general(jnp.ones((1, DIM), jnp.float32), r * r,
                             (((1,), (1,)), ((), ())))
    qr = jax.lax.dot_general(q, r, (((1,), (1,)), ((), ())),
                             preferred_element_type=jnp.float32)
    d = qn + rn - 2.0 * qr
    iota = jax.lax.broadcasted_iota(jnp.int32, d.shape, 1)
    cols = []
    for _ in range(K):
        g = jnp.min(d, axis=1, keepdims=True)
        am = jnp.min(jnp.where(d == g, iota, NR), axis=1, keepdims=True)
        cols.append(am)
        d = jnp.where(iota == am, jnp.inf, d)
    o_ref[0] = jnp.concatenate(cols, axis=1)


def _knn(q, r, K):
    Bq, M, _ = q.shape
    NR = r.shape[1]
    TM = 128 if NR >= 2048 else min(M, 512)
    return pl.pallas_call(
        functools.partial(_knn_body, K, NR),
        grid=(Bq, M // TM),
        in_specs=[pl.BlockSpec((1, TM, DIM), lambda b, i: (b, i, 0)),
                  pl.BlockSpec((1, NR, DIM), lambda b, i: (b, 0, 0))],
        out_specs=pl.BlockSpec((1, TM, K), lambda b, i: (b, i, 0)),
        out_shape=jax.ShapeDtypeStruct((Bq, M, K), jnp.int32),
    )(q, r)


def _gather(a, idx):
    return jax.vmap(lambda ab, ib: ab[ib])(a, idx)


def _ptconv(p, x, pin, pout, idx, K):
    feats = _gather(x, idx)
    pts = _gather(pin, idx) - pout[:, :, None, :]
    maxi = jnp.sqrt(jax.lax.stop_gradient((pts ** 2).sum(-1)).max(-1))
    maxi = jnp.where(maxi == 0.0, 1.0, maxi)
    pts = pts / maxi[:, :, None, None]
    d = pts[..., None] - p["c"][None, None, None, :, :]
    d = d.reshape(d.shape[0], d.shape[1], d.shape[2], DIM * NC)
    d = jax.nn.relu(d @ p["l1w"] + p["l1b"])
    d = jax.nn.relu(d @ p["l2w"] + p["l2b"])
    d = jax.nn.relu(d @ p["l3w"] + p["l3b"])
    r = jnp.einsum("bmkc,bmkn->bmcn", feats, d)
    r = r.reshape(r.shape[0], r.shape[1], -1)
    return (r @ p["w"].reshape(-1, p["w"].shape[2])) / K


def _bn(x, bp, eps=1e-5):
    m = x.mean(axis=(0, 1))
    v = x.var(axis=(0, 1))
    return bp["g"] * (x - m) / jnp.sqrt(v + eps) + bp["b"]


def _fc_kernel(x_ref, w_ref, b_ref, o_ref):
    o_ref[...] = jnp.dot(x_ref[...], w_ref[...],
                         preferred_element_type=jnp.float32) + b_ref[...]


def _fc(x2d, w, b):
    R = x2d.shape[0]
    return pl.pallas_call(
        _fc_kernel,
        out_shape=jax.ShapeDtypeStruct((R, w.shape[1]), jnp.float32),
        in_specs=[pl.BlockSpec((R, x2d.shape[1]), lambda: (0, 0)),
                  pl.BlockSpec(w.shape, lambda: (0, 0)),
                  pl.BlockSpec((1, w.shape[1]), lambda: (0, 0))],
        out_specs=pl.BlockSpec((R, w.shape[1]), lambda: (0, 0)),
    )(x2d, w, b.reshape(1, -1))


def kernel(x, input_pts, params):
    pts = _pts_pyramid(input_pts)
    idx = {name: _knn(pts[o], pts[r], K) for (name, r, o, K) in KSPEC}

    def cv(name, xin, r, o, K):
        return jax.nn.relu(_bn(
            _ptconv(params[name], xin, pts[r], pts[o], idx[name], K),
            params["bn_" + name]))

    x0 = cv("cv0", x, 0, 0, 16)
    x1 = cv("cv1", x0, 0, 1, 16)
    x2 = cv("cv2", x1, 1, 2, 16)
    x3 = cv("cv3", x2, 2, 3, 16)
    x4 = cv("cv4", x3, 3, 4, 8)
    x5 = cv("cv5", x4, 4, 5, 8)
    x6 = cv("cv6", x5, 5, 6, 4)
    x5d = jnp.concatenate([cv("cv5d", x6, 6, 5, 4), x5], axis=2)
    x4d = jnp.concatenate([cv("cv4d", x5d, 5, 4, 4), x4], axis=2)
    x3d = jnp.concatenate([cv("cv3d", x4d, 4, 3, 4), x3], axis=2)
    x2d = jnp.concatenate([cv("cv2d", x3d, 3, 2, 8), x2], axis=2)
    x1d = jnp.concatenate([cv("cv1d", x2d, 2, 1, 8), x1], axis=2)
    x0d = jnp.concatenate([cv("cv0d", x1d, 1, 0, 8), x0], axis=2)

    xout = _fc(x0d.reshape(-1, x0d.shape[2]), params["fcout_w"],
               params["fcout_b"])
    xout = xout.reshape(x.shape[0], -1, xout.shape[1])
    return (xout, x0d)
